# Initial kernel scaffold; baseline (speedup 1.0000x reference)
import jax
import jax.numpy as jnp
from jax.experimental import pallas as pl


def kernel(pred, target):
    raise NotImplementedError("write your pallas kernel here")


# speedup vs baseline: 3.3326x; 3.3326x over previous
import jax
import jax.numpy as jnp
from jax.experimental import pallas as pl
from jax.experimental.pallas import tpu as pltpu

_EPS = 1e-8
_M = 256
_C = (_M + 1) / 2.0
_NEG = -3e38
_POS = 3e38


def _soft_rank_row(x):
    m = _M
    jcol = jax.lax.broadcasted_iota(jnp.int32, (m, m), 1)
    irow = jax.lax.broadcasted_iota(jnp.int32, (m, m), 0)
    xr = x[None, :]
    xc = x[:, None]
    gt = (xr > xc).astype(jnp.float32)
    tie = ((xr == xc) & (jcol < irow)).astype(jnp.float32)
    pos = jnp.sum(gt + tie, axis=1)
    kf = jcol.astype(jnp.float32)
    oh = (pos[:, None] == kf).astype(jnp.float32)
    s = jnp.sum(oh * xc, axis=0)
    w = (jnp.float32(m)
         - jax.lax.broadcasted_iota(jnp.int32, (m,), 0).astype(jnp.float32))
    y = s - w
    upper = (irow <= jcol).astype(jnp.float32)
    cs = jnp.sum(upper * y[:, None], axis=0)
    csj = cs - y
    lengths = kf - irow.astype(jnp.float32) + 1.0
    means = (cs[None, :] - csj[:, None]) / lengths
    a = jnp.where(jcol >= irow, means, _NEG)
    sh = 1
    while sh < m:
        shifted = jnp.concatenate(
            [a[:, sh:], jnp.full((m, sh), _NEG, jnp.float32)], axis=1)
        a = jnp.maximum(a, shifted)
        sh *= 2
    sh = 1
    while sh < m:
        shifted = jnp.concatenate(
            [jnp.full((sh, m), _POS, jnp.float32), a[:-sh, :]], axis=0)
        a = jnp.minimum(a, shifted)
        sh *= 2
    v = jnp.sum(a * (jcol == irow).astype(jnp.float32), axis=0)
    out_sorted = s - v
    return jnp.sum(oh * out_sorted[None, :], axis=1)


def _body(pred_ref, targ_ref, out_ref, acc_ref):
    i = pl.program_id(0)
    j = pl.program_id(1)

    @pl.when((i == 0) & (j == 0))
    def _():
        for k in range(5):
            acc_ref[k] = jnp.float32(0.0)

    rp = _soft_rank_row(pred_ref[j, :]) - _C
    rt = _soft_rank_row(targ_ref[j, :]) - _C
    acc_ref[0] += jnp.sum(rp)
    acc_ref[1] += jnp.sum(rp * rp)
    acc_ref[2] += jnp.sum(rt)
    acc_ref[3] += jnp.sum(rt * rt)
    acc_ref[4] += jnp.sum(rp * rt)

    @pl.when((i == _M // 8 - 1) & (j == 7))
    def _():
        n = jnp.float32(_M * _M)
        sp, spp = acc_ref[0], acc_ref[1]
        st, stt = acc_ref[2], acc_ref[3]
        spt = acc_ref[4]
        varp = spp - sp * sp / n
        vart = stt - st * st / n
        cov = spt - sp * st / n
        denom = (jnp.sqrt(varp) + _EPS) * (jnp.sqrt(vart) + _EPS)
        out_ref[0, 0] = 1.0 - cov / denom


def kernel(pred, target):
    out = pl.pallas_call(
        _body,
        grid=(_M // 8, 8),
        in_specs=[
            pl.BlockSpec((8, _M), lambda i, j: (i, 0)),
            pl.BlockSpec((8, _M), lambda i, j: (i, 0)),
        ],
        out_specs=pl.BlockSpec(memory_space=pltpu.SMEM),
        out_shape=jax.ShapeDtypeStruct((1, 1), jnp.float32),
        scratch_shapes=[pltpu.SMEM((8,), jnp.float32)],
    )(pred, target)
    return out[0, 0]


# speedup vs baseline: 39.1207x; 11.7389x over previous
import functools

import jax
import jax.numpy as jnp
from jax import lax
from jax.experimental import pallas as pl
from jax.experimental.pallas import tpu as pltpu
from jax.experimental.pallas import tpu_sc as plsc

_EPS = 1e-8
_M = 256
_NROW = 256
_C = (_M + 1) / 2.0
_NC = 2
_NS = 16
_NW = _NC * _NS
_RPW = _NROW // _NW
_L = 16


def _iota16():
    return lax.broadcasted_iota(jnp.int32, (_L,), 0)


def _cmp_exchange(keys, vals, a, b, desc):
    ka, kb = keys[a], keys[b]
    va, vb = vals[a], vals[b]
    m = (ka >= kb) if desc else (ka <= kb)
    keys[a] = jnp.where(m, ka, kb)
    keys[b] = jnp.where(m, kb, ka)
    vals[a] = jnp.where(m, va, vb)
    vals[b] = jnp.where(m, vb, va)


def _sort_row_desc(keys, vals):
    nv = _M // _L
    for j in range(nv):
        keys[j], vals[j] = plsc.sort_key_val(
            keys[j], vals[j], descending=(j % 2 == 0))
    for size in (2, 4, 8, 16):
        for base in range(0, nv, size):
            desc = ((base // size) % 2 == 0)
            d = size // 2
            while d >= 1:
                for off in range(0, size, 2 * d):
                    for i in range(d):
                        _cmp_exchange(keys, vals, base + off + i,
                                      base + off + i + d, desc)
                d //= 2
            for j in range(base, base + size):
                keys[j], vals[j] = plsc.sort_key_val(
                    keys[j], vals[j], descending=desc)


def _sc_body(pred_hbm, targ_hbm, out_hbm,
             rows, sk, perm, sums, cnts, outb, pvec):
    c = lax.axis_index("c")
    s = lax.axis_index("s")
    wid = s * _NC + c
    lane = _iota16()
    lane_f = lane.astype(jnp.float32)

    pltpu.sync_copy(pred_hbm.at[pl.ds(wid * _RPW, _RPW)],
                    rows.at[pl.ds(0, _RPW)])
    pltpu.sync_copy(targ_hbm.at[pl.ds(wid * _RPW, _RPW)],
                    rows.at[pl.ds(_RPW, _RPW)])

    def sort_one(r, _):
        keys = [rows[r, pl.ds(j * _L, _L)] for j in range(_M // _L)]
        vals = [lane + j * _L for j in range(_M // _L)]
        _sort_row_desc(keys, vals)
        for j in range(_M // _L):
            idx = j * _M + lane * _L + r
            plsc.store_scatter(sk, [idx], keys[j])
            plsc.store_scatter(perm, [idx], vals[j])
        return 0

    lax.fori_loop(0, _L, sort_one, 0)

    def probe(cur_sum, cur_cnt, depth):
        m0 = depth > 0
        gidx = jnp.where(m0, (depth - 1) * _L + lane, lane)
        ps = plsc.load_gather(sums, [gidx], mask=m0)
        pc = plsc.load_gather(cnts, [gidx], mask=m0)
        viol = m0 & (cur_sum * pc >= ps * cur_cnt)
        return ps, pc, viol

    def pav_step(t, depth):
        y_t = (sk[pl.ds(t * _L, _L)]
               - (jnp.float32(_M) - t.astype(jnp.float32)))
        cur_sum = y_t
        cur_cnt = jnp.ones((_L,), jnp.float32)
        ps, pc, viol = probe(cur_sum, cur_cnt, depth)

        def wcond(st):
            return jnp.sum(st[5].astype(jnp.int32)) > 0

        def wbody(st):
            cur_sum, cur_cnt, depth, ps, pc, viol = st
            cur_sum = cur_sum + jnp.where(viol, ps, 0.0)
            cur_cnt = cur_cnt + jnp.where(viol, pc, 0.0)
            depth = depth - viol.astype(jnp.int32)
            ps, pc, viol = probe(cur_sum, cur_cnt, depth)
            return cur_sum, cur_cnt, depth, ps, pc, viol

        cur_sum, cur_cnt, depth, _, _, _ = lax.while_loop(
            wcond, wbody, (cur_sum, cur_cnt, depth, ps, pc, viol))
        pidx = depth * _L + lane
        plsc.store_scatter(sums, [pidx], cur_sum)
        plsc.store_scatter(cnts, [pidx], cur_cnt)
        return depth + 1

    lax.fori_loop(0, _M, pav_step, jnp.zeros((_L,), jnp.int32))

    off = jnp.where(lane < _RPW, lane, _M * _RPW + lane - _RPW)

    def expand_step(t, carry):
        b, rem, mean, ssum, ssq = carry
        sk_t = sk[pl.ds(t * _L, _L)]
        perm_t = perm[pl.ds(t * _L, _L)]
        need = rem <= 0.0
        b = b + need.astype(jnp.int32)
        gidx = b * _L + lane
        gs = plsc.load_gather(sums, [gidx], mask=need)
        gc = plsc.load_gather(cnts, [gidx], mask=need)
        mean = jnp.where(need, gs / gc, mean)
        rem = jnp.where(need, gc, rem)
        out_c = sk_t - mean - jnp.float32(_C)
        rem = rem - 1.0
        plsc.store_scatter(outb, [perm_t * _RPW + off], out_c)
        return b, rem, mean, ssum + out_c, ssq + out_c * out_c

    init = (jnp.full((_L,), -1, jnp.int32), jnp.zeros((_L,), jnp.float32),
            jnp.zeros((_L,), jnp.float32), jnp.zeros((_L,), jnp.float32),
            jnp.zeros((_L,), jnp.float32))
    _, _, _, ssum, ssq = lax.fori_loop(0, _M, expand_step, init)

    def prod_step(t, pacc):
        op = outb[pl.ds(t * _L, _L)]
        ot = outb[pl.ds(_M * _RPW + t * _L, _L)]
        return pacc + op * ot

    pacc = lax.fori_loop(0, _M * _RPW // _L, prod_step,
                         jnp.zeros((_L,), jnp.float32))

    is_p = lane < _RPW
    zero = jnp.zeros((_L,), jnp.float32)
    sp = jnp.sum(jnp.where(is_p, ssum, zero))
    st = jnp.sum(jnp.where(is_p, zero, ssum))
    spp = jnp.sum(jnp.where(is_p, ssq, zero))
    stt = jnp.sum(jnp.where(is_p, zero, ssq))
    spt = jnp.sum(pacc)
    res = (sp * (lane_f == 0.0).astype(jnp.float32)
           + spp * (lane_f == 1.0).astype(jnp.float32)
           + st * (lane_f == 2.0).astype(jnp.float32)
           + stt * (lane_f == 3.0).astype(jnp.float32)
           + spt * (lane_f == 4.0).astype(jnp.float32))
    pvec[...] = res
    pltpu.sync_copy(pvec, out_hbm.at[wid])


def _combine_body(p_ref, out_ref):
    x = p_ref[...]
    n = jnp.float32(_M * _NROW)
    sp = jnp.sum(x[:, 0])
    spp = jnp.sum(x[:, 1])
    st = jnp.sum(x[:, 2])
    stt = jnp.sum(x[:, 3])
    spt = jnp.sum(x[:, 4])
    varp = spp - sp * sp / n
    vart = stt - st * st / n
    cov = spt - sp * st / n
    denom = (jnp.sqrt(varp) + _EPS) * (jnp.sqrt(vart) + _EPS)
    out_ref[0, 0] = 1.0 - cov / denom


def kernel(pred, target):
    mesh = plsc.VectorSubcoreMesh(core_axis_name="c", subcore_axis_name="s",
                                  num_cores=_NC, num_subcores=_NS)
    sc = pl.kernel(
        _sc_body,
        out_type=jax.ShapeDtypeStruct((_NW, _L), jnp.float32),
        mesh=mesh,
        compiler_params=pltpu.CompilerParams(needs_layout_passes=False),
        scratch_types=[
            pltpu.VMEM((_L, _M), jnp.float32),
            pltpu.VMEM((_M * _L,), jnp.float32),
            pltpu.VMEM((_M * _L,), jnp.int32),
            pltpu.VMEM((_M * _L,), jnp.float32),
            pltpu.VMEM((_M * _L,), jnp.float32),
            pltpu.VMEM((2 * _M * _RPW,), jnp.float32),
            pltpu.VMEM((_L,), jnp.float32),
        ],
    )
    partials = sc(pred, target)
    out = pl.pallas_call(
        _combine_body,
        in_specs=[pl.BlockSpec((_NW, _L), lambda: (0, 0))],
        out_specs=pl.BlockSpec(memory_space=pltpu.SMEM),
        out_shape=jax.ShapeDtypeStruct((1, 1), jnp.float32),
    )(partials)
    return out[0, 0]


# speedup vs baseline: 39.1850x; 1.0016x over previous
import functools

import jax
import jax.numpy as jnp
from jax import lax
from jax.experimental import pallas as pl
from jax.experimental.pallas import tpu as pltpu
from jax.experimental.pallas import tpu_sc as plsc

_EPS = 1e-8
_M = 256
_NROW = 256
_C = (_M + 1) / 2.0
_NC = 2
_NS = 16
_NW = _NC * _NS
_RPW = _NROW // _NW
_L = 16


def _iota16():
    return lax.broadcasted_iota(jnp.int32, (_L,), 0)


def _cmp_exchange(keys, vals, a, b, desc):
    ka, kb = keys[a], keys[b]
    va, vb = vals[a], vals[b]
    m = (ka >= kb) if desc else (ka <= kb)
    keys[a] = jnp.where(m, ka, kb)
    keys[b] = jnp.where(m, kb, ka)
    vals[a] = jnp.where(m, va, vb)
    vals[b] = jnp.where(m, vb, va)


def _sort_row_desc(keys, vals):
    nv = _M // _L
    for j in range(nv):
        keys[j], vals[j] = plsc.sort_key_val(
            keys[j], vals[j], descending=(j % 2 == 0))
    for size in (2, 4, 8, 16):
        for base in range(0, nv, size):
            desc = ((base // size) % 2 == 0)
            d = size // 2
            while d >= 1:
                for off in range(0, size, 2 * d):
                    for i in range(d):
                        _cmp_exchange(keys, vals, base + off + i,
                                      base + off + i + d, desc)
                d //= 2
            for j in range(base, base + size):
                keys[j], vals[j] = plsc.sort_key_val(
                    keys[j], vals[j], descending=desc)


def _sc_body(pred_hbm, targ_hbm, out_hbm,
             rows, sk, perm, sums, cnts, outb, pvec):
    c = lax.axis_index("c")
    s = lax.axis_index("s")
    wid = s * _NC + c
    lane = _iota16()
    lane_f = lane.astype(jnp.float32)

    with jax.named_scope("sc_load"):
        pltpu.sync_copy(pred_hbm.at[pl.ds(wid * _RPW, _RPW)],
                        rows.at[pl.ds(0, _RPW)])
        pltpu.sync_copy(targ_hbm.at[pl.ds(wid * _RPW, _RPW)],
                        rows.at[pl.ds(_RPW, _RPW)])

    def sort_one(r, _):
        keys = [rows[r, pl.ds(j * _L, _L)] for j in range(_M // _L)]
        vals = [lane + j * _L for j in range(_M // _L)]
        _sort_row_desc(keys, vals)
        for j in range(_M // _L):
            idx = j * _M + lane * _L + r
            plsc.store_scatter(sk, [idx], keys[j])
            plsc.store_scatter(perm, [idx], vals[j])
        return 0

    with jax.named_scope("sc_sort"):
        lax.fori_loop(0, _L, sort_one, 0)

    def probe(cur_sum, cur_cnt, depth):
        m0 = depth > 0
        gidx = jnp.where(m0, (depth - 1) * _L + lane, lane)
        ps = plsc.load_gather(sums, [gidx], mask=m0)
        pc = plsc.load_gather(cnts, [gidx], mask=m0)
        viol = m0 & (cur_sum * pc >= ps * cur_cnt)
        return ps, pc, viol

    def pav_step(t, depth):
        y_t = (sk[pl.ds(t * _L, _L)]
               - (jnp.float32(_M) - t.astype(jnp.float32)))
        cur_sum = y_t
        cur_cnt = jnp.ones((_L,), jnp.float32)
        ps, pc, viol = probe(cur_sum, cur_cnt, depth)

        def wcond(st):
            return jnp.sum(st[5].astype(jnp.int32)) > 0

        def wbody(st):
            cur_sum, cur_cnt, depth, ps, pc, viol = st
            cur_sum = cur_sum + jnp.where(viol, ps, 0.0)
            cur_cnt = cur_cnt + jnp.where(viol, pc, 0.0)
            depth = depth - viol.astype(jnp.int32)
            ps, pc, viol = probe(cur_sum, cur_cnt, depth)
            return cur_sum, cur_cnt, depth, ps, pc, viol

        cur_sum, cur_cnt, depth, _, _, _ = lax.while_loop(
            wcond, wbody, (cur_sum, cur_cnt, depth, ps, pc, viol))
        pidx = depth * _L + lane
        plsc.store_scatter(sums, [pidx], cur_sum)
        plsc.store_scatter(cnts, [pidx], cur_cnt)
        return depth + 1

    with jax.named_scope("sc_pav"):
        lax.fori_loop(0, _M, pav_step, jnp.zeros((_L,), jnp.int32))

    off = jnp.where(lane < _RPW, lane, _M * _RPW + lane - _RPW)

    def expand_step(t, carry):
        b, rem, mean, ssum, ssq = carry
        sk_t = sk[pl.ds(t * _L, _L)]
        perm_t = perm[pl.ds(t * _L, _L)]
        need = rem <= 0.0
        b = b + need.astype(jnp.int32)
        gidx = b * _L + lane
        gs = plsc.load_gather(sums, [gidx], mask=need)
        gc = plsc.load_gather(cnts, [gidx], mask=need)
        mean = jnp.where(need, gs / gc, mean)
        rem = jnp.where(need, gc, rem)
        out_c = sk_t - mean - jnp.float32(_C)
        rem = rem - 1.0
        plsc.store_scatter(outb, [perm_t * _RPW + off], out_c)
        return b, rem, mean, ssum + out_c, ssq + out_c * out_c

    init = (jnp.full((_L,), -1, jnp.int32), jnp.zeros((_L,), jnp.float32),
            jnp.zeros((_L,), jnp.float32), jnp.zeros((_L,), jnp.float32),
            jnp.zeros((_L,), jnp.float32))
    with jax.named_scope("sc_expand"):
        _, _, _, ssum, ssq = lax.fori_loop(0, _M, expand_step, init)

    def prod_step(t, pacc):
        op = outb[pl.ds(t * _L, _L)]
        ot = outb[pl.ds(_M * _RPW + t * _L, _L)]
        return pacc + op * ot

    with jax.named_scope("sc_prod"):
        pacc = lax.fori_loop(0, _M * _RPW // _L, prod_step,
                             jnp.zeros((_L,), jnp.float32))

    is_p = lane < _RPW
    zero = jnp.zeros((_L,), jnp.float32)
    sp = jnp.sum(jnp.where(is_p, ssum, zero))
    st = jnp.sum(jnp.where(is_p, zero, ssum))
    spp = jnp.sum(jnp.where(is_p, ssq, zero))
    stt = jnp.sum(jnp.where(is_p, zero, ssq))
    spt = jnp.sum(pacc)
    res = (sp * (lane_f == 0.0).astype(jnp.float32)
           + spp * (lane_f == 1.0).astype(jnp.float32)
           + st * (lane_f == 2.0).astype(jnp.float32)
           + stt * (lane_f == 3.0).astype(jnp.float32)
           + spt * (lane_f == 4.0).astype(jnp.float32))
    pvec[...] = res
    pltpu.sync_copy(pvec, out_hbm.at[wid])


def _combine_body(p_ref, out_ref):
    x = p_ref[...]
    n = jnp.float32(_M * _NROW)
    sp = jnp.sum(x[:, 0])
    spp = jnp.sum(x[:, 1])
    st = jnp.sum(x[:, 2])
    stt = jnp.sum(x[:, 3])
    spt = jnp.sum(x[:, 4])
    varp = spp - sp * sp / n
    vart = stt - st * st / n
    cov = spt - sp * st / n
    denom = (jnp.sqrt(varp) + _EPS) * (jnp.sqrt(vart) + _EPS)
    out_ref[0, 0] = 1.0 - cov / denom


def kernel(pred, target):
    mesh = plsc.VectorSubcoreMesh(core_axis_name="c", subcore_axis_name="s",
                                  num_cores=_NC, num_subcores=_NS)
    sc = pl.kernel(
        _sc_body,
        out_type=jax.ShapeDtypeStruct((_NW, _L), jnp.float32),
        mesh=mesh,
        compiler_params=pltpu.CompilerParams(needs_layout_passes=False),
        scratch_types=[
            pltpu.VMEM((_L, _M), jnp.float32),
            pltpu.VMEM((_M * _L,), jnp.float32),
            pltpu.VMEM((_M * _L,), jnp.int32),
            pltpu.VMEM((_M * _L,), jnp.float32),
            pltpu.VMEM((_M * _L,), jnp.float32),
            pltpu.VMEM((2 * _M * _RPW,), jnp.float32),
            pltpu.VMEM((_L,), jnp.float32),
        ],
    )
    partials = sc(pred, target)
    out = pl.pallas_call(
        _combine_body,
        in_specs=[pl.BlockSpec((_NW, _L), lambda: (0, 0))],
        out_specs=pl.BlockSpec(memory_space=pltpu.SMEM),
        out_shape=jax.ShapeDtypeStruct((1, 1), jnp.float32),
    )(partials)
    return out[0, 0]


# speedup vs baseline: 48.4761x; 1.2371x over previous
import functools

import jax
import jax.numpy as jnp
from jax import lax
from jax.experimental import pallas as pl
from jax.experimental.pallas import tpu as pltpu
from jax.experimental.pallas import tpu_sc as plsc

_EPS = 1e-8
_M = 256
_NROW = 256
_C = (_M + 1) / 2.0
_NC = 2
_NS = 16
_NW = _NC * _NS
_RPW = _NROW // _NW
_L = 16


def _iota16():
    return lax.broadcasted_iota(jnp.int32, (_L,), 0)


def _cmp_exchange(keys, vals, a, b, desc):
    ka, kb = keys[a], keys[b]
    va, vb = vals[a], vals[b]
    m = (ka >= kb) if desc else (ka <= kb)
    keys[a] = jnp.where(m, ka, kb)
    keys[b] = jnp.where(m, kb, ka)
    vals[a] = jnp.where(m, va, vb)
    vals[b] = jnp.where(m, vb, va)


def _sort_row_desc(keys, vals):
    nv = _M // _L
    for j in range(nv):
        keys[j], vals[j] = plsc.sort_key_val(
            keys[j], vals[j], descending=(j % 2 == 0))
    for size in (2, 4, 8, 16):
        for base in range(0, nv, size):
            desc = ((base // size) % 2 == 0)
            d = size // 2
            while d >= 1:
                for off in range(0, size, 2 * d):
                    for i in range(d):
                        _cmp_exchange(keys, vals, base + off + i,
                                      base + off + i + d, desc)
                d //= 2
            for j in range(base, base + size):
                keys[j], vals[j] = plsc.sort_key_val(
                    keys[j], vals[j], descending=desc)


def _sc_body(pred_hbm, targ_hbm, out_hbm,
             rows, sk, perm, sums, cnts, outb, pvec):
    c = lax.axis_index("c")
    s = lax.axis_index("s")
    wid = s * _NC + c
    lane = _iota16()
    lane_f = lane.astype(jnp.float32)

    with jax.named_scope("sc_load"):
        pltpu.sync_copy(pred_hbm.at[pl.ds(wid * _RPW, _RPW)],
                        rows.at[pl.ds(0, _RPW)])
        pltpu.sync_copy(targ_hbm.at[pl.ds(wid * _RPW, _RPW)],
                        rows.at[pl.ds(_RPW, _RPW)])

    with jax.named_scope("sc_sort"):
        @plsc.parallel_loop(0, _L, unroll=2)
        def _sort_loop(r):
            keys = [rows[r, pl.ds(j * _L, _L)] for j in range(_M // _L)]
            vals = [lane + j * _L for j in range(_M // _L)]
            _sort_row_desc(keys, vals)
            for j in range(_M // _L):
                idx = j * _M + lane * _L + r
                plsc.store_scatter(sk, [idx], keys[j])
                plsc.store_scatter(perm, [idx], vals[j])

    def pav_step(it, st):
        cur_sum, cur_cnt, depth, tpos = st
        tsafe = jnp.minimum(tpos, _M - 1)
        ynext = plsc.load_gather(sk, [tsafe * _L + lane])
        ynext = ynext - (jnp.float32(_M) - tsafe.astype(jnp.float32))
        m0 = depth > 0
        gidx = jnp.where(m0, (depth - 1) * _L + lane, lane)
        ps = plsc.load_gather(sums, [gidx], mask=m0)
        pc = plsc.load_gather(cnts, [gidx], mask=m0)
        viol = m0 & (cur_sum * pc >= ps * cur_cnt)
        cur_sum = cur_sum + jnp.where(viol, ps, 0.0)
        cur_cnt = cur_cnt + jnp.where(viol, pc, 0.0)
        depth = depth - viol.astype(jnp.int32)
        pushm = (~viol) & (tpos < _M)
        pidx = depth * _L + lane
        plsc.store_scatter(sums, [pidx], cur_sum, mask=pushm)
        plsc.store_scatter(cnts, [pidx], cur_cnt, mask=pushm)
        depth = depth + pushm.astype(jnp.int32)
        cur_sum = jnp.where(pushm, ynext, cur_sum)
        cur_cnt = jnp.where(pushm, 1.0, cur_cnt)
        tpos = tpos + pushm.astype(jnp.int32)
        return cur_sum, cur_cnt, depth, tpos

    with jax.named_scope("sc_pav"):
        y0 = sk[pl.ds(0, _L)] - jnp.float32(_M)
        init_pav = (y0, jnp.ones((_L,), jnp.float32),
                    jnp.zeros((_L,), jnp.int32), jnp.ones((_L,), jnp.int32))
        cur_sum, cur_cnt, depth, _ = lax.fori_loop(
            0, 2 * _M - 2, pav_step, init_pav)
        pidx = depth * _L + lane
        plsc.store_scatter(sums, [pidx], cur_sum)
        plsc.store_scatter(cnts, [pidx], cur_cnt)

    off = jnp.where(lane < _RPW, lane, _M * _RPW + lane - _RPW)

    def expand_step(t, carry):
        b, rem, mean, ssum, ssq = carry
        sk_t = sk[pl.ds(t * _L, _L)]
        perm_t = perm[pl.ds(t * _L, _L)]
        need = rem <= 0.0
        b = b + need.astype(jnp.int32)
        gidx = b * _L + lane
        gs = plsc.load_gather(sums, [gidx], mask=need)
        gc = plsc.load_gather(cnts, [gidx], mask=need)
        mean = jnp.where(need, gs / gc, mean)
        rem = jnp.where(need, gc, rem)
        out_c = sk_t - mean - jnp.float32(_C)
        rem = rem - 1.0
        plsc.store_scatter(outb, [perm_t * _RPW + off], out_c)
        return b, rem, mean, ssum + out_c, ssq + out_c * out_c

    init = (jnp.full((_L,), -1, jnp.int32), jnp.zeros((_L,), jnp.float32),
            jnp.zeros((_L,), jnp.float32), jnp.zeros((_L,), jnp.float32),
            jnp.zeros((_L,), jnp.float32))
    with jax.named_scope("sc_expand"):
        _, _, _, ssum, ssq = lax.fori_loop(0, _M, expand_step, init)

    def prod_step(t, pacc):
        op = outb[pl.ds(t * _L, _L)]
        ot = outb[pl.ds(_M * _RPW + t * _L, _L)]
        return pacc + op * ot

    with jax.named_scope("sc_prod"):
        pacc = lax.fori_loop(0, _M * _RPW // _L, prod_step,
                             jnp.zeros((_L,), jnp.float32))

    is_p = lane < _RPW
    zero = jnp.zeros((_L,), jnp.float32)
    sp = jnp.sum(jnp.where(is_p, ssum, zero))
    st = jnp.sum(jnp.where(is_p, zero, ssum))
    spp = jnp.sum(jnp.where(is_p, ssq, zero))
    stt = jnp.sum(jnp.where(is_p, zero, ssq))
    spt = jnp.sum(pacc)
    res = (sp * (lane_f == 0.0).astype(jnp.float32)
           + spp * (lane_f == 1.0).astype(jnp.float32)
           + st * (lane_f == 2.0).astype(jnp.float32)
           + stt * (lane_f == 3.0).astype(jnp.float32)
           + spt * (lane_f == 4.0).astype(jnp.float32))
    pvec[...] = res
    pltpu.sync_copy(pvec, out_hbm.at[wid])


def _combine_body(p_ref, out_ref):
    x = p_ref[...]
    n = jnp.float32(_M * _NROW)
    sp = jnp.sum(x[:, 0])
    spp = jnp.sum(x[:, 1])
    st = jnp.sum(x[:, 2])
    stt = jnp.sum(x[:, 3])
    spt = jnp.sum(x[:, 4])
    varp = spp - sp * sp / n
    vart = stt - st * st / n
    cov = spt - sp * st / n
    denom = (jnp.sqrt(varp) + _EPS) * (jnp.sqrt(vart) + _EPS)
    out_ref[0, 0] = 1.0 - cov / denom


def kernel(pred, target):
    mesh = plsc.VectorSubcoreMesh(core_axis_name="c", subcore_axis_name="s",
                                  num_cores=_NC, num_subcores=_NS)
    sc = pl.kernel(
        _sc_body,
        out_type=jax.ShapeDtypeStruct((_NW, _L), jnp.float32),
        mesh=mesh,
        compiler_params=pltpu.CompilerParams(needs_layout_passes=False),
        scratch_types=[
            pltpu.VMEM((_L, _M), jnp.float32),
            pltpu.VMEM((_M * _L,), jnp.float32),
            pltpu.VMEM((_M * _L,), jnp.int32),
            pltpu.VMEM((_M * _L,), jnp.float32),
            pltpu.VMEM((_M * _L,), jnp.float32),
            pltpu.VMEM((2 * _M * _RPW,), jnp.float32),
            pltpu.VMEM((_L,), jnp.float32),
        ],
    )
    partials = sc(pred, target)
    out = pl.pallas_call(
        _combine_body,
        in_specs=[pl.BlockSpec((_NW, _L), lambda: (0, 0))],
        out_specs=pl.BlockSpec(memory_space=pltpu.SMEM),
        out_shape=jax.ShapeDtypeStruct((1, 1), jnp.float32),
    )(partials)
    return out[0, 0]


# speedup vs baseline: 54.2395x; 1.1189x over previous
import functools

import jax
import jax.numpy as jnp
from jax import lax
from jax.experimental import pallas as pl
from jax.experimental.pallas import tpu as pltpu
from jax.experimental.pallas import tpu_sc as plsc

_EPS = 1e-8
_M = 256
_NROW = 256
_C = (_M + 1) / 2.0
_NC = 2
_NS = 16
_NW = _NC * _NS
_RPW = _NROW // _NW
_L = 16


def _iota16():
    return lax.broadcasted_iota(jnp.int32, (_L,), 0)


def _cmp_exchange(keys, vals, a, b, desc):
    ka, kb = keys[a], keys[b]
    va, vb = vals[a], vals[b]
    m = (ka >= kb) if desc else (ka <= kb)
    keys[a] = jnp.where(m, ka, kb)
    keys[b] = jnp.where(m, kb, ka)
    vals[a] = jnp.where(m, va, vb)
    vals[b] = jnp.where(m, vb, va)


def _sort_row_desc(keys, vals):
    nv = _M // _L
    for j in range(nv):
        keys[j], vals[j] = plsc.sort_key_val(
            keys[j], vals[j], descending=(j % 2 == 0))
    for size in (2, 4, 8, 16):
        for base in range(0, nv, size):
            desc = ((base // size) % 2 == 0)
            d = size // 2
            while d >= 1:
                for off in range(0, size, 2 * d):
                    for i in range(d):
                        _cmp_exchange(keys, vals, base + off + i,
                                      base + off + i + d, desc)
                d //= 2
            for j in range(base, base + size):
                keys[j], vals[j] = plsc.sort_key_val(
                    keys[j], vals[j], descending=desc)


def _sc_body(pred_hbm, targ_hbm, out_hbm,
             rows, sk, perm, sums, cnts, outb, pvec):
    c = lax.axis_index("c")
    s = lax.axis_index("s")
    wid = s * _NC + c
    lane = _iota16()
    lane_f = lane.astype(jnp.float32)

    with jax.named_scope("sc_load"):
        pltpu.sync_copy(pred_hbm.at[pl.ds(wid * _RPW, _RPW)],
                        rows.at[pl.ds(0, _RPW)])
        pltpu.sync_copy(targ_hbm.at[pl.ds(wid * _RPW, _RPW)],
                        rows.at[pl.ds(_RPW, _RPW)])

    with jax.named_scope("sc_sort"):
        @plsc.parallel_loop(0, _L, unroll=2)
        def _sort_loop(r):
            keys = [rows[r, pl.ds(j * _L, _L)] for j in range(_M // _L)]
            vals = [lane + j * _L for j in range(_M // _L)]
            _sort_row_desc(keys, vals)
            for j in range(_M // _L):
                idx = j * _M + lane * _L + r
                plsc.store_scatter(sk, [idx], keys[j])
                plsc.store_scatter(perm, [idx], vals[j])

    def pav_step(it, st):
        (cur_sum, cur_cnt, prev_sum, prev_cnt,
         p2_sum, p2_cnt, depth, tpos) = st
        tsafe = jnp.minimum(tpos, _M - 1)
        ynext = plsc.load_gather(sk, [tsafe * _L + lane])
        ynext = ynext - (jnp.float32(_M) - tsafe.astype(jnp.float32))
        viol = (depth > 0) & (cur_sum * prev_cnt >= prev_sum * cur_cnt)
        msum = cur_sum + jnp.where(viol, prev_sum, 0.0)
        mcnt = cur_cnt + jnp.where(viol, prev_cnt, 0.0)
        gm = viol & (depth >= 3)
        gidx = jnp.where(gm, (depth - 3) * _L + lane, lane)
        gs = plsc.load_gather(sums, [gidx], mask=gm)
        gc = plsc.load_gather(cnts, [gidx], mask=gm)
        depth2 = depth - viol.astype(jnp.int32)
        pushm = (~viol) & (tpos < _M)
        pidx = depth2 * _L + lane
        plsc.store_scatter(sums, [pidx], msum, mask=pushm)
        plsc.store_scatter(cnts, [pidx], mcnt, mask=pushm)
        p2s = jnp.where(pushm, prev_sum, jnp.where(viol, gs, p2_sum))
        p2c = jnp.where(pushm, prev_cnt, jnp.where(viol, gc, p2_cnt))
        prs = jnp.where(pushm, msum, jnp.where(viol, p2_sum, prev_sum))
        prc = jnp.where(pushm, mcnt, jnp.where(viol, p2_cnt, prev_cnt))
        depth3 = depth2 + pushm.astype(jnp.int32)
        cs = jnp.where(pushm, ynext, msum)
        cc = jnp.where(pushm, 1.0, mcnt)
        tpos = tpos + pushm.astype(jnp.int32)
        return cs, cc, prs, prc, p2s, p2c, depth3, tpos

    with jax.named_scope("sc_pav"):
        y0 = sk[pl.ds(0, _L)] - jnp.float32(_M)
        zf = jnp.zeros((_L,), jnp.float32)
        init_pav = (y0, jnp.ones((_L,), jnp.float32), zf, zf, zf, zf,
                    jnp.zeros((_L,), jnp.int32), jnp.ones((_L,), jnp.int32))
        cur_sum, cur_cnt, _, _, _, _, depth, _ = lax.fori_loop(
            0, 2 * _M - 2, pav_step, init_pav)
        pidx = depth * _L + lane
        plsc.store_scatter(sums, [pidx], cur_sum)
        plsc.store_scatter(cnts, [pidx], cur_cnt)

    off = jnp.where(lane < _RPW, lane, _M * _RPW + lane - _RPW)

    def expand_step(t, carry):
        b, rem, mean, nmean, nrem, ssum, ssq = carry
        sk_t = sk[pl.ds(t * _L, _L)]
        perm_t = perm[pl.ds(t * _L, _L)]
        need = rem <= 0.0
        mean = jnp.where(need, nmean, mean)
        rem = jnp.where(need, nrem, rem)
        b = b + need.astype(jnp.int32)
        gidx = jnp.minimum(b + 1, _M - 1) * _L + lane
        gs = plsc.load_gather(sums, [gidx], mask=need)
        gc = plsc.load_gather(cnts, [gidx], mask=need)
        nmean = jnp.where(need, gs / gc, nmean)
        nrem = jnp.where(need, gc, nrem)
        out_c = sk_t - mean - jnp.float32(_C)
        rem = rem - 1.0
        plsc.store_scatter(outb, [perm_t * _RPW + off], out_c)
        return b, rem, mean, nmean, nrem, ssum + out_c, ssq + out_c * out_c

    with jax.named_scope("sc_expand"):
        s0 = sums[pl.ds(0, _L)]
        c0 = cnts[pl.ds(0, _L)]
        s1 = sums[pl.ds(_L, _L)]
        c1 = cnts[pl.ds(_L, _L)]
        zf32 = jnp.zeros((_L,), jnp.float32)
        init = (jnp.zeros((_L,), jnp.int32), c0, s0 / c0,
                s1 / c1, c1, zf32, zf32)
        _, _, _, _, _, ssum, ssq = lax.fori_loop(0, _M, expand_step, init)

    def prod_step(t, pacc):
        op = outb[pl.ds(t * _L, _L)]
        ot = outb[pl.ds(_M * _RPW + t * _L, _L)]
        return pacc + op * ot

    with jax.named_scope("sc_prod"):
        pacc = lax.fori_loop(0, _M * _RPW // _L, prod_step,
                             jnp.zeros((_L,), jnp.float32))

    is_p = lane < _RPW
    zero = jnp.zeros((_L,), jnp.float32)
    sp = jnp.sum(jnp.where(is_p, ssum, zero))
    st = jnp.sum(jnp.where(is_p, zero, ssum))
    spp = jnp.sum(jnp.where(is_p, ssq, zero))
    stt = jnp.sum(jnp.where(is_p, zero, ssq))
    spt = jnp.sum(pacc)
    res = (sp * (lane_f == 0.0).astype(jnp.float32)
           + spp * (lane_f == 1.0).astype(jnp.float32)
           + st * (lane_f == 2.0).astype(jnp.float32)
           + stt * (lane_f == 3.0).astype(jnp.float32)
           + spt * (lane_f == 4.0).astype(jnp.float32))
    pvec[...] = res
    pltpu.sync_copy(pvec, out_hbm.at[wid])


def _combine_body(p_ref, out_ref):
    x = p_ref[...]
    n = jnp.float32(_M * _NROW)
    sp = jnp.sum(x[:, 0])
    spp = jnp.sum(x[:, 1])
    st = jnp.sum(x[:, 2])
    stt = jnp.sum(x[:, 3])
    spt = jnp.sum(x[:, 4])
    varp = spp - sp * sp / n
    vart = stt - st * st / n
    cov = spt - sp * st / n
    denom = (jnp.sqrt(varp) + _EPS) * (jnp.sqrt(vart) + _EPS)
    out_ref[0, 0] = 1.0 - cov / denom


def kernel(pred, target):
    mesh = plsc.VectorSubcoreMesh(core_axis_name="c", subcore_axis_name="s",
                                  num_cores=_NC, num_subcores=_NS)
    sc = pl.kernel(
        _sc_body,
        out_type=jax.ShapeDtypeStruct((_NW, _L), jnp.float32),
        mesh=mesh,
        compiler_params=pltpu.CompilerParams(needs_layout_passes=False),
        scratch_types=[
            pltpu.VMEM((_L, _M), jnp.float32),
            pltpu.VMEM((_M * _L,), jnp.float32),
            pltpu.VMEM((_M * _L,), jnp.int32),
            pltpu.VMEM((_M * _L,), jnp.float32),
            pltpu.VMEM((_M * _L,), jnp.float32),
            pltpu.VMEM((2 * _M * _RPW,), jnp.float32),
            pltpu.VMEM((_L,), jnp.float32),
        ],
    )
    partials = sc(pred, target)
    out = pl.pallas_call(
        _combine_body,
        in_specs=[pl.BlockSpec((_NW, _L), lambda: (0, 0))],
        out_specs=pl.BlockSpec(memory_space=pltpu.SMEM),
        out_shape=jax.ShapeDtypeStruct((1, 1), jnp.float32),
    )(partials)
    return out[0, 0]
